# baseline (speedup 1.0000x reference)
import functools

import jax
import jax.numpy as jnp
from jax import lax
from jax.experimental import pallas as pl
from jax.experimental.pallas import tpu as pltpu
from jax.experimental.pallas import tpu_sc as plsc

EMB_DIM = 64
PAD_DIM = 128
NUM_WORKERS = 32
NBUF = 4
CHUNK = 200


def _emb_body(idx_hbm, tab_hbm, out_hbm, *refs, rows_per_worker):
    idxv = refs[0:NBUF]
    rowsv = refs[NBUF:2 * NBUF]
    si = refs[2 * NBUF:3 * NBUF]
    sg = refs[3 * NBUF:4 * NBUF]
    so = refs[4 * NBUF:5 * NBUF]

    wid = lax.axis_index("s") * 2 + lax.axis_index("c")
    base = wid * rows_per_worker
    n = rows_per_worker // CHUNK

    def idx_slice(i):
        return idx_hbm.at[pl.ds(base + i * CHUNK, CHUNK)]

    def out_slice(i):
        return out_hbm.at[pl.ds(base + i * CHUNK, CHUNK)]

    def start_idx(i, b):
        pltpu.async_copy(idx_slice(i), idxv[b], si[b])

    def wait_idx(b):
        pltpu.make_async_copy(idx_slice(0), idxv[b], si[b]).wait()

    def start_gather(b):
        pltpu.async_copy(tab_hbm.at[idxv[b]], rowsv[b], sg[b])

    def wait_gather(b):
        pltpu.make_async_copy(tab_hbm.at[idxv[b]], rowsv[b], sg[b]).wait()

    def start_wb(i, b):
        pltpu.async_copy(rowsv[b], out_slice(i), so[b])

    def wait_wb(b):
        pltpu.make_async_copy(rowsv[b], out_slice(0), so[b]).wait()

    def step(i, b, do_wb_wait, do_gather, do_idx):
        wait_gather(b)
        start_wb(i, b)
        if do_gather:
            b2 = (b + 2) % NBUF
            wait_idx(b2)
            if do_wb_wait:
                wait_wb(b2)
            start_gather(b2)
        if do_idx:
            start_idx(i + NBUF, b)

    pltpu.sync_copy(idx_slice(0), idxv[0])
    for k in range(1, NBUF):
        start_idx(k, k)
    start_gather(0)
    wait_idx(1)
    start_gather(1)

    for i in range(NBUF):
        step(i, i % NBUF, i + 2 >= NBUF, i + 2 < n, i + NBUF < n)

    def body(j, carry):
        for b in range(NBUF):
            step(NBUF * j + b, b, True, True, True)
        return carry

    lax.fori_loop(1, n // NBUF - 1, body, 0)

    for i in range(n - NBUF, n):
        step(i, i % NBUF, i + 2 >= NBUF, i + 2 < n, i + NBUF < n)

    for b in range(NBUF):
        wait_wb(b)


def kernel(x, table):
    b, h = x.shape
    n = b * h
    rows_per_worker = n // NUM_WORKERS
    idx = x.reshape(n).astype(jnp.int32)
    tab_pad = jnp.concatenate([table, table], axis=1)

    mesh = plsc.VectorSubcoreMesh(core_axis_name="c", subcore_axis_name="s")
    emb = functools.partial(
        pl.kernel,
        mesh=mesh,
        out_type=jax.ShapeDtypeStruct((n, PAD_DIM), jnp.float32),
        scratch_types=(
            [pltpu.VMEM((CHUNK,), jnp.int32) for _ in range(NBUF)]
            + [pltpu.VMEM((CHUNK, PAD_DIM), jnp.float32) for _ in range(NBUF)]
            + [pltpu.SemaphoreType.DMA for _ in range(3 * NBUF)]
        ),
    )(functools.partial(_emb_body, rows_per_worker=rows_per_worker))

    out = emb(idx, tab_pad)
    return out[:, :EMB_DIM].reshape(b, h, EMB_DIM)


# speedup vs baseline: 1.1489x; 1.1489x over previous
import functools

import jax
import jax.numpy as jnp
from jax import lax
from jax.experimental import pallas as pl
from jax.experimental.pallas import tpu as pltpu
from jax.experimental.pallas import tpu_sc as plsc

EMB_DIM = 64
PAD_DIM = 128
NUM_WORKERS = 32
NBUF = 4
CHUNK = 200


def _emb_body(idx_hbm, tab_hbm, out_hbm, *refs, rows_per_worker):
    idxv = refs[0:NBUF]
    rowsv = refs[NBUF:2 * NBUF]
    si = refs[2 * NBUF:3 * NBUF]
    sg = refs[3 * NBUF:4 * NBUF]
    so = refs[4 * NBUF:5 * NBUF]

    wid = lax.axis_index("s") * 2 + lax.axis_index("c")
    base = wid * rows_per_worker
    n = rows_per_worker // CHUNK

    def idx_slice(i):
        return idx_hbm.at[pl.ds(base + i * CHUNK, CHUNK)]

    def out_slice(i):
        return out_hbm.at[pl.ds(base + i * CHUNK, CHUNK)]

    def start_idx(i, b):
        pltpu.async_copy(idx_slice(i), idxv[b], si[b])

    def wait_idx(b):
        pltpu.make_async_copy(idx_slice(0), idxv[b], si[b]).wait()

    def start_gather(b):
        pltpu.async_copy(tab_hbm.at[idxv[b]], rowsv[b], sg[b])

    def wait_gather(b):
        pltpu.make_async_copy(tab_hbm.at[idxv[b]], rowsv[b], sg[b]).wait()

    def start_wb(i, b):
        pltpu.async_copy(rowsv[b], out_slice(i), so[b])

    def wait_wb(b):
        pltpu.make_async_copy(rowsv[b], out_slice(0), so[b]).wait()

    def step(i, b, do_wb_wait, do_gather, do_idx):
        wait_gather(b)
        start_wb(i, b)
        if do_gather:
            b2 = (b + 2) % NBUF
            wait_idx(b2)
            if do_wb_wait:
                wait_wb(b2)
            start_gather(b2)
        if do_idx:
            start_idx(i + NBUF, b)

    pltpu.sync_copy(idx_slice(0), idxv[0])
    for k in range(1, NBUF):
        start_idx(k, k)
    start_gather(0)
    wait_idx(1)
    start_gather(1)

    for i in range(NBUF):
        step(i, i % NBUF, i + 2 >= NBUF, i + 2 < n, i + NBUF < n)

    def body(j, carry):
        for b in range(NBUF):
            step(NBUF * j + b, b, True, True, True)
        return carry

    lax.fori_loop(1, n // NBUF - 1, body, 0)

    for i in range(n - NBUF, n):
        step(i, i % NBUF, i + 2 >= NBUF, i + 2 < n, i + NBUF < n)

    for b in range(NBUF):
        wait_wb(b)


def kernel(x, table):
    b, h = x.shape
    n = b * h
    rows_per_worker = n // NUM_WORKERS
    idx = x.reshape(n).astype(jnp.int32)
    tab_pad = jnp.pad(table, ((0, 0), (0, PAD_DIM - EMB_DIM)))

    mesh = plsc.VectorSubcoreMesh(core_axis_name="c", subcore_axis_name="s")
    emb = functools.partial(
        pl.kernel,
        mesh=mesh,
        out_type=jax.ShapeDtypeStruct((n, PAD_DIM), jnp.float32),
        scratch_types=(
            [pltpu.VMEM((CHUNK,), jnp.int32) for _ in range(NBUF)]
            + [pltpu.VMEM((CHUNK, PAD_DIM), jnp.float32) for _ in range(NBUF)]
            + [pltpu.SemaphoreType.DMA for _ in range(3 * NBUF)]
        ),
    )(functools.partial(_emb_body, rows_per_worker=rows_per_worker))

    out = emb(idx, tab_pad)
    return out[:, :EMB_DIM].reshape(b, h, EMB_DIM)
